# baseline (speedup 1.0000x reference)
import functools

import jax
import jax.numpy as jnp
from jax import lax
from jax.experimental import pallas as pl
from jax.experimental.pallas import tpu as pltpu
from jax.experimental.pallas import tpu_sc as plsc

N = 10000
E = 320000
D = 128
NP = 10240
NC = 2
NS = 16
NW = NC * NS
CK = 88
CH = 120
NBUF = 4
NIDX = 8
EW = CH * CK
EPAD = NW * EW
RPT = NP // NS
EPS = 1e-5

_PIECES = [(i * CK, CK) for i in range(RPT // CK)]
if RPT % CK:
  _PIECES.append((RPT - RPT % CK, RPT % CK))


@functools.cache
def _sc_degree_kernel():
  mesh = plsc.VectorSubcoreMesh(core_axis_name="c", subcore_axis_name="s")

  @functools.partial(
      pl.kernel,
      out_type=jax.ShapeDtypeStruct((NC, NP), jnp.float32),
      mesh=mesh,
      scratch_types=[
          pltpu.VMEM_SHARED((NP,), jnp.float32),
          pltpu.VMEM((CH, 2, CK), jnp.int32),
          pltpu.VMEM((128,), jnp.float32),
          pltpu.VMEM((RPT,), jnp.float32),
          pltpu.SemaphoreType.DMA,
      ]
      + [pltpu.SemaphoreType.DMA for _ in range(8)],
  )
  def _sc_degree(e_hbm, out_hbm, acc, idx_v, ones_v, buf_v, isem, *ssem):
    c = lax.axis_index("c")
    s = lax.axis_index("s")
    wid = s * NC + c
    pltpu.async_copy(e_hbm.at[wid], idx_v, isem)

    one16 = jnp.ones((16,), jnp.float32)
    zero16 = jnp.zeros((16,), jnp.float32)

    @pl.loop(0, 128, step=16)
    def _(i):
      ones_v.at[pl.ds(i, 16)][...] = one16

    @pl.loop(0, RPT, step=16)
    def _(i):
      buf_v.at[pl.ds(i, 16)][...] = zero16

    pltpu.sync_copy(buf_v, acc.at[pl.ds(s * RPT, RPT)])
    pltpu.make_async_copy(e_hbm.at[wid], idx_v, isem).wait()
    plsc.subcore_barrier()

    ones = ones_v.at[pl.ds(0, CK)]

    @pl.loop(0, CH, step=8)
    def _(j):
      for t in range(8):
        @pl.when(j > 0)
        def _():
          pltpu.make_async_copy(ones, acc.at[pl.ds(0, CK)], ssem[t]).wait()

        pltpu.async_copy(ones, acc.at[idx_v.at[j + t, 1]], ssem[t], add=True)

    for t in range(8):
      pltpu.make_async_copy(ones, acc.at[pl.ds(0, CK)], ssem[t]).wait()

    plsc.subcore_barrier()
    pltpu.sync_copy(acc.at[pl.ds(s * RPT, RPT)], buf_v)
    pltpu.sync_copy(buf_v, out_hbm.at[c, pl.ds(s * RPT, RPT)])

  return _sc_degree


@functools.cache
def _sc_aggregate_kernel():
  mesh = plsc.VectorSubcoreMesh(core_axis_name="c", subcore_axis_name="s")

  @functools.partial(
      pl.kernel,
      out_type=jax.ShapeDtypeStruct((NC, NP, D), jnp.float32),
      mesh=mesh,
      scratch_types=[pltpu.VMEM_SHARED((NP, D), jnp.float32)]
      + [pltpu.VMEM((CK, D), jnp.float32) for _ in range(NBUF)]
      + [pltpu.VMEM((2, CK), jnp.int32) for _ in range(NIDX)]
      + [pltpu.SemaphoreType.DMA for _ in range(2 * NBUF + NIDX + 1)],
  )
  def _sc_aggregate(y_hbm, e_hbm, out_hbm, acc, *scr):
    rows = scr[:NBUF]
    slots = scr[NBUF:NBUF + NIDX]
    gsem = scr[NBUF + NIDX:2 * NBUF + NIDX]
    ssem = scr[2 * NBUF + NIDX:3 * NBUF + NIDX]
    isem = scr[3 * NBUF + NIDX:3 * NBUF + 2 * NIDX]
    zsem = scr[-1]
    c = lax.axis_index("c")
    s = lax.axis_index("s")
    wid = s * NC + c

    zero16 = jnp.zeros((16,), jnp.float32)

    @pl.loop(0, CK)
    def _(r):
      @pl.loop(0, D, step=16)
      def _(q):
        rows[0].at[r, pl.ds(q, 16)][...] = zero16

    for off, ln in _PIECES:
      pltpu.async_copy(rows[0].at[pl.ds(0, ln)],
                       acc.at[pl.ds(s * RPT + off, ln)], zsem)

    for k in range(NIDX):
      pltpu.async_copy(e_hbm.at[wid, k], slots[k], isem[k])
    for off, ln in _PIECES:
      pltpu.make_async_copy(rows[0].at[pl.ds(0, ln)],
                            acc.at[pl.ds(s * RPT, ln)], zsem).wait()
    for k in range(NBUF):
      pltpu.make_async_copy(e_hbm.at[wid, k], slots[k], isem[k]).wait()

    for b in range(NBUF):
      pltpu.async_copy(y_hbm.at[slots[b].at[0]], rows[b], gsem[b])

    plsc.subcore_barrier()

    @pl.loop(0, CH, step=2 * NBUF)
    def _(j):
      for t in range(NBUF):
        b = t
        pltpu.make_async_copy(y_hbm.at[slots[t].at[0]], rows[b],
                              gsem[b]).wait()
        pltpu.async_copy(rows[b], acc.at[slots[t].at[1]], ssem[b], add=True)
      for t in range(NBUF):
        b = t
        pltpu.make_async_copy(rows[b], acc.at[slots[t].at[1]], ssem[b]).wait()

        @pl.when(j + NIDX + t < CH)
        def _():
          pltpu.async_copy(e_hbm.at[wid, j + NIDX + t], slots[t], isem[t])

        pltpu.make_async_copy(e_hbm.at[wid, 0], slots[t + NBUF],
                              isem[t + NBUF]).wait()
        pltpu.async_copy(y_hbm.at[slots[t + NBUF].at[0]], rows[b], gsem[b])
      for t in range(NBUF, 2 * NBUF):
        b = t - NBUF
        pltpu.make_async_copy(y_hbm.at[slots[t].at[0]], rows[b],
                              gsem[b]).wait()
        pltpu.async_copy(rows[b], acc.at[slots[t].at[1]], ssem[b], add=True)
      for t in range(NBUF, 2 * NBUF):
        b = t - NBUF
        pltpu.make_async_copy(rows[b], acc.at[slots[t].at[1]], ssem[b]).wait()

        @pl.when(j + NIDX + t < CH)
        def _():
          pltpu.async_copy(e_hbm.at[wid, j + NIDX + t], slots[t], isem[t])

        @pl.when(j + NIDX + b < CH)
        def _():
          pltpu.make_async_copy(e_hbm.at[wid, 0], slots[b], isem[b]).wait()
          pltpu.async_copy(y_hbm.at[slots[b].at[0]], rows[b], gsem[b])

    plsc.subcore_barrier()

    pltpu.async_copy(acc.at[pl.ds(s * RPT, RPT)],
                     out_hbm.at[c, pl.ds(s * RPT, RPT)], gsem[0])
    pltpu.make_async_copy(acc.at[pl.ds(s * RPT, RPT)],
                          out_hbm.at[c, pl.ds(s * RPT, RPT)], gsem[0]).wait()

  return _sc_aggregate


def _tc_first_body(x_ref, w_ref, degp_ref, y_ref, dinv_ref):
    deg = degp_ref[0, 0:N, :] + degp_ref[1, 0:N, :] + 1.0
    dinv = lax.rsqrt(deg)
    xw = jnp.dot(x_ref[...], w_ref[...], preferred_element_type=jnp.float32)
    y_ref[0:N, :] = dinv * xw
    y_ref[N:NP, :] = jnp.zeros((NP - N, D), jnp.float32)
    dinv_ref[...] = dinv


def _tc_first(x, w1, degp):
    return pl.pallas_call(
        _tc_first_body,
        out_shape=(
            jax.ShapeDtypeStruct((NP, D), jnp.float32),
            jax.ShapeDtypeStruct((N, 1), jnp.float32),
        ),
    )(x, w1, degp)


def _bn_core(sp_ref, y_ref, dinv_ref, b_ref, g_ref, be_ref):
    s = sp_ref[0, 0:N, :] + sp_ref[1, 0:N, :]
    dinv = dinv_ref[...]
    t = dinv * (s + y_ref[0:N, :]) + b_ref[...]
    m = jnp.mean(t, axis=0, keepdims=True)
    v = jnp.mean(t * t, axis=0, keepdims=True) - m * m
    return g_ref[...] * (t - m) * lax.rsqrt(v + EPS) + be_ref[...]


def _tc_mid_body(sp_ref, y_ref, dinv_ref, b_ref, g_ref, be_ref, w_ref, yn_ref):
    h = jnp.maximum(_bn_core(sp_ref, y_ref, dinv_ref, b_ref, g_ref, be_ref), 0.0)
    xw = jnp.dot(h, w_ref[...], preferred_element_type=jnp.float32)
    yn_ref[0:N, :] = dinv_ref[...] * xw
    yn_ref[N:NP, :] = jnp.zeros((NP - N, D), jnp.float32)


def _tc_mid(sp, y, dinv, b, g, be, w_next):
    return pl.pallas_call(
        _tc_mid_body,
        out_shape=jax.ShapeDtypeStruct((NP, D), jnp.float32),
    )(sp, y, dinv, b.reshape(1, D), g.reshape(1, D), be.reshape(1, D), w_next)


def _tc_last_body(sp_ref, y_ref, dinv_ref, b_ref, g_ref, be_ref, o_ref):
    o_ref[...] = _bn_core(sp_ref, y_ref, dinv_ref, b_ref, g_ref, be_ref)


def _tc_last(sp, y, dinv, b, g, be):
    return pl.pallas_call(
        _tc_last_body,
        out_shape=jax.ShapeDtypeStruct((N, D), jnp.float32),
    )(sp, y, dinv, b.reshape(1, D), g.reshape(1, D), be.reshape(1, D))


def kernel(x, edge_index, W1, b1, g1, be1, W2, b2, g2, be2,
           W3, b3, g3, be3, W4, b4, g4, be4):
    ei = edge_index.astype(jnp.int32)
    npad = EPAD - E
    pad_idx = N + (jnp.arange(npad, dtype=jnp.int32) % (NP - N))
    src = jnp.concatenate([ei[0], pad_idx])
    dst = jnp.concatenate([ei[1], pad_idx])
    edges = jnp.stack(
        [src.reshape(NW, CH, CK), dst.reshape(NW, CH, CK)], axis=2)

    _sc_degree = _sc_degree_kernel()
    _sc_aggregate = _sc_aggregate_kernel()
    degp = _sc_degree(edges).reshape(NC, NP, 1)
    y, dinv = _tc_first(x, W1, degp)
    sp = _sc_aggregate(y, edges)
    y = _tc_mid(sp, y, dinv, b1, g1, be1, W2)
    sp = _sc_aggregate(y, edges)
    y = _tc_mid(sp, y, dinv, b2, g2, be2, W3)
    sp = _sc_aggregate(y, edges)
    y = _tc_mid(sp, y, dinv, b3, g3, be3, W4)
    sp = _sc_aggregate(y, edges)
    return _tc_last(sp, y, dinv, b4, g4, be4)


# speedup vs baseline: 1.0405x; 1.0405x over previous
import functools

import jax
import jax.numpy as jnp
from jax import lax
from jax.experimental import pallas as pl
from jax.experimental.pallas import tpu as pltpu
from jax.experimental.pallas import tpu_sc as plsc

N = 10000
E = 320000
D = 128
NP = 10240
NC = 2
NS = 16
NW = NC * NS
CK = 72
CH = 140
NBUF = 5
NIDX = 10
EW = CH * CK
EPAD = NW * EW
RPT = NP // NS
EPS = 1e-5

_PIECES = [(i * CK, CK) for i in range(RPT // CK)]
if RPT % CK:
  _PIECES.append((RPT - RPT % CK, RPT % CK))


@functools.cache
def _sc_degree_kernel():
  mesh = plsc.VectorSubcoreMesh(core_axis_name="c", subcore_axis_name="s")

  @functools.partial(
      pl.kernel,
      out_type=jax.ShapeDtypeStruct((NC, NP), jnp.float32),
      mesh=mesh,
      scratch_types=[
          pltpu.VMEM_SHARED((NP,), jnp.float32),
          pltpu.VMEM((CH, 2, CK), jnp.int32),
          pltpu.VMEM((128,), jnp.float32),
          pltpu.VMEM((RPT,), jnp.float32),
          pltpu.SemaphoreType.DMA,
      ]
      + [pltpu.SemaphoreType.DMA for _ in range(10)],
  )
  def _sc_degree(e_hbm, out_hbm, acc, idx_v, ones_v, buf_v, isem, *ssem):
    c = lax.axis_index("c")
    s = lax.axis_index("s")
    wid = s * NC + c
    pltpu.async_copy(e_hbm.at[wid], idx_v, isem)

    one16 = jnp.ones((16,), jnp.float32)
    zero16 = jnp.zeros((16,), jnp.float32)

    @pl.loop(0, 128, step=16)
    def _(i):
      ones_v.at[pl.ds(i, 16)][...] = one16

    @pl.loop(0, RPT, step=16)
    def _(i):
      buf_v.at[pl.ds(i, 16)][...] = zero16

    pltpu.sync_copy(buf_v, acc.at[pl.ds(s * RPT, RPT)])
    pltpu.make_async_copy(e_hbm.at[wid], idx_v, isem).wait()
    plsc.subcore_barrier()

    ones = ones_v.at[pl.ds(0, CK)]

    @pl.loop(0, CH, step=10)
    def _(j):
      for t in range(10):
        @pl.when(j > 0)
        def _():
          pltpu.make_async_copy(ones, acc.at[pl.ds(0, CK)], ssem[t]).wait()

        pltpu.async_copy(ones, acc.at[idx_v.at[j + t, 1]], ssem[t], add=True)

    for t in range(10):
      pltpu.make_async_copy(ones, acc.at[pl.ds(0, CK)], ssem[t]).wait()

    plsc.subcore_barrier()
    pltpu.sync_copy(acc.at[pl.ds(s * RPT, RPT)], buf_v)
    pltpu.sync_copy(buf_v, out_hbm.at[c, pl.ds(s * RPT, RPT)])

  return _sc_degree


@functools.cache
def _sc_aggregate_kernel():
  mesh = plsc.VectorSubcoreMesh(core_axis_name="c", subcore_axis_name="s")

  @functools.partial(
      pl.kernel,
      out_type=jax.ShapeDtypeStruct((NC, NP, D), jnp.float32),
      mesh=mesh,
      scratch_types=[pltpu.VMEM_SHARED((NP, D), jnp.float32)]
      + [pltpu.VMEM((CK, D), jnp.float32) for _ in range(NBUF)]
      + [pltpu.VMEM((2, CK), jnp.int32) for _ in range(NIDX)]
      + [pltpu.SemaphoreType.DMA for _ in range(2 * NBUF + NIDX + 1)],
  )
  def _sc_aggregate(y_hbm, e_hbm, out_hbm, acc, *scr):
    rows = scr[:NBUF]
    slots = scr[NBUF:NBUF + NIDX]
    gsem = scr[NBUF + NIDX:2 * NBUF + NIDX]
    ssem = scr[2 * NBUF + NIDX:3 * NBUF + NIDX]
    isem = scr[3 * NBUF + NIDX:3 * NBUF + 2 * NIDX]
    zsem = scr[-1]
    c = lax.axis_index("c")
    s = lax.axis_index("s")
    wid = s * NC + c

    zero16 = jnp.zeros((16,), jnp.float32)

    @pl.loop(0, CK)
    def _(r):
      @pl.loop(0, D, step=16)
      def _(q):
        rows[0].at[r, pl.ds(q, 16)][...] = zero16

    for off, ln in _PIECES:
      pltpu.async_copy(rows[0].at[pl.ds(0, ln)],
                       acc.at[pl.ds(s * RPT + off, ln)], zsem)

    for k in range(NIDX):
      pltpu.async_copy(e_hbm.at[wid, k], slots[k], isem[k])
    for off, ln in _PIECES:
      pltpu.make_async_copy(rows[0].at[pl.ds(0, ln)],
                            acc.at[pl.ds(s * RPT, ln)], zsem).wait()
    for k in range(NBUF):
      pltpu.make_async_copy(e_hbm.at[wid, k], slots[k], isem[k]).wait()

    for b in range(NBUF):
      pltpu.async_copy(y_hbm.at[slots[b].at[0]], rows[b], gsem[b])

    plsc.subcore_barrier()

    @pl.loop(0, CH, step=2 * NBUF)
    def _(j):
      for t in range(NBUF):
        b = t
        pltpu.make_async_copy(y_hbm.at[slots[t].at[0]], rows[b],
                              gsem[b]).wait()
        pltpu.async_copy(rows[b], acc.at[slots[t].at[1]], ssem[b], add=True)
      for t in range(NBUF):
        b = t
        pltpu.make_async_copy(rows[b], acc.at[slots[t].at[1]], ssem[b]).wait()

        @pl.when(j + NIDX + t < CH)
        def _():
          pltpu.async_copy(e_hbm.at[wid, j + NIDX + t], slots[t], isem[t])

        pltpu.make_async_copy(e_hbm.at[wid, 0], slots[t + NBUF],
                              isem[t + NBUF]).wait()
        pltpu.async_copy(y_hbm.at[slots[t + NBUF].at[0]], rows[b], gsem[b])
      for t in range(NBUF, 2 * NBUF):
        b = t - NBUF
        pltpu.make_async_copy(y_hbm.at[slots[t].at[0]], rows[b],
                              gsem[b]).wait()
        pltpu.async_copy(rows[b], acc.at[slots[t].at[1]], ssem[b], add=True)
      for t in range(NBUF, 2 * NBUF):
        b = t - NBUF
        pltpu.make_async_copy(rows[b], acc.at[slots[t].at[1]], ssem[b]).wait()

        @pl.when(j + NIDX + t < CH)
        def _():
          pltpu.async_copy(e_hbm.at[wid, j + NIDX + t], slots[t], isem[t])

        @pl.when(j + NIDX + b < CH)
        def _():
          pltpu.make_async_copy(e_hbm.at[wid, 0], slots[b], isem[b]).wait()
          pltpu.async_copy(y_hbm.at[slots[b].at[0]], rows[b], gsem[b])

    plsc.subcore_barrier()

    for i, (off, ln) in enumerate(_PIECES):
      b = i % 2
      if i >= 2:
        po, pl_ = _PIECES[i - 2]
        pltpu.make_async_copy(rows[b].at[pl.ds(0, pl_)],
                              out_hbm.at[c, pl.ds(s * RPT, pl_)],
                              gsem[b]).wait()
      pltpu.sync_copy(acc.at[pl.ds(s * RPT + off, ln)],
                      rows[b].at[pl.ds(0, ln)])
      pltpu.async_copy(rows[b].at[pl.ds(0, ln)],
                       out_hbm.at[c, pl.ds(s * RPT + off, ln)], gsem[b])
    for i in range(max(0, len(_PIECES) - 2), len(_PIECES)):
      b = i % 2
      off, ln = _PIECES[i]
      pltpu.make_async_copy(rows[b].at[pl.ds(0, ln)],
                            out_hbm.at[c, pl.ds(s * RPT, ln)],
                            gsem[b]).wait()

  return _sc_aggregate


def _tc_first_body(x_ref, w_ref, degp_ref, y_ref, dinv_ref):
    deg = degp_ref[0, 0:N, :] + degp_ref[1, 0:N, :] + 1.0
    dinv = lax.rsqrt(deg)
    xw = jnp.dot(x_ref[...], w_ref[...], preferred_element_type=jnp.float32)
    y_ref[0:N, :] = dinv * xw
    y_ref[N:NP, :] = jnp.zeros((NP - N, D), jnp.float32)
    dinv_ref[...] = dinv


def _tc_first(x, w1, degp):
    return pl.pallas_call(
        _tc_first_body,
        out_shape=(
            jax.ShapeDtypeStruct((NP, D), jnp.float32),
            jax.ShapeDtypeStruct((N, 1), jnp.float32),
        ),
    )(x, w1, degp)


def _bn_core(sp_ref, y_ref, dinv_ref, b_ref, g_ref, be_ref):
    s = sp_ref[0, 0:N, :] + sp_ref[1, 0:N, :]
    dinv = dinv_ref[...]
    t = dinv * (s + y_ref[0:N, :]) + b_ref[...]
    m = jnp.mean(t, axis=0, keepdims=True)
    v = jnp.mean(t * t, axis=0, keepdims=True) - m * m
    return g_ref[...] * (t - m) * lax.rsqrt(v + EPS) + be_ref[...]


def _tc_mid_body(sp_ref, y_ref, dinv_ref, b_ref, g_ref, be_ref, w_ref, yn_ref):
    h = jnp.maximum(_bn_core(sp_ref, y_ref, dinv_ref, b_ref, g_ref, be_ref), 0.0)
    xw = jnp.dot(h, w_ref[...], preferred_element_type=jnp.float32)
    yn_ref[0:N, :] = dinv_ref[...] * xw
    yn_ref[N:NP, :] = jnp.zeros((NP - N, D), jnp.float32)


def _tc_mid(sp, y, dinv, b, g, be, w_next):
    return pl.pallas_call(
        _tc_mid_body,
        out_shape=jax.ShapeDtypeStruct((NP, D), jnp.float32),
    )(sp, y, dinv, b.reshape(1, D), g.reshape(1, D), be.reshape(1, D), w_next)


def _tc_last_body(sp_ref, y_ref, dinv_ref, b_ref, g_ref, be_ref, o_ref):
    o_ref[...] = _bn_core(sp_ref, y_ref, dinv_ref, b_ref, g_ref, be_ref)


def _tc_last(sp, y, dinv, b, g, be):
    return pl.pallas_call(
        _tc_last_body,
        out_shape=jax.ShapeDtypeStruct((N, D), jnp.float32),
    )(sp, y, dinv, b.reshape(1, D), g.reshape(1, D), be.reshape(1, D))


def kernel(x, edge_index, W1, b1, g1, be1, W2, b2, g2, be2,
           W3, b3, g3, be3, W4, b4, g4, be4):
    ei = edge_index.astype(jnp.int32)
    npad = EPAD - E
    pad_idx = N + (jnp.arange(npad, dtype=jnp.int32) % (NP - N))
    src = jnp.concatenate([ei[0], pad_idx])
    dst = jnp.concatenate([ei[1], pad_idx])
    edges = jnp.stack(
        [src.reshape(NW, CH, CK), dst.reshape(NW, CH, CK)], axis=2)

    _sc_degree = _sc_degree_kernel()
    _sc_aggregate = _sc_aggregate_kernel()
    degp = _sc_degree(edges).reshape(NC, NP, 1)
    y, dinv = _tc_first(x, W1, degp)
    sp = _sc_aggregate(y, edges)
    y = _tc_mid(sp, y, dinv, b1, g1, be1, W2)
    sp = _sc_aggregate(y, edges)
    y = _tc_mid(sp, y, dinv, b2, g2, be2, W3)
    sp = _sc_aggregate(y, edges)
    y = _tc_mid(sp, y, dinv, b3, g3, be3, W4)
    sp = _sc_aggregate(y, edges)
    return _tc_last(sp, y, dinv, b4, g4, be4)


# speedup vs baseline: 1.0426x; 1.0020x over previous
import functools

import jax
import jax.numpy as jnp
from jax import lax
from jax.experimental import pallas as pl
from jax.experimental.pallas import tpu as pltpu
from jax.experimental.pallas import tpu_sc as plsc

N = 10000
E = 320000
D = 128
NP = 10240
NC = 2
NS = 16
NW = NC * NS
CK = 56
CH = 180
NBUF = 6
NIDX = 12
EW = CH * CK
EPAD = NW * EW
RPT = NP // NS
EPS = 1e-5

_PIECES = [(i * CK, CK) for i in range(RPT // CK)]
if RPT % CK:
  _PIECES.append((RPT - RPT % CK, RPT % CK))


@functools.cache
def _sc_degree_kernel():
  mesh = plsc.VectorSubcoreMesh(core_axis_name="c", subcore_axis_name="s")

  @functools.partial(
      pl.kernel,
      out_type=jax.ShapeDtypeStruct((NC, NP), jnp.float32),
      mesh=mesh,
      scratch_types=[
          pltpu.VMEM_SHARED((NP,), jnp.float32),
          pltpu.VMEM((CH, 2, CK), jnp.int32),
          pltpu.VMEM((128,), jnp.float32),
          pltpu.VMEM((RPT,), jnp.float32),
          pltpu.SemaphoreType.DMA,
      ]
      + [pltpu.SemaphoreType.DMA for _ in range(10)],
  )
  def _sc_degree(e_hbm, out_hbm, acc, idx_v, ones_v, buf_v, isem, *ssem):
    c = lax.axis_index("c")
    s = lax.axis_index("s")
    wid = s * NC + c
    pltpu.async_copy(e_hbm.at[wid], idx_v, isem)

    one16 = jnp.ones((16,), jnp.float32)
    zero16 = jnp.zeros((16,), jnp.float32)

    @pl.loop(0, 128, step=16)
    def _(i):
      ones_v.at[pl.ds(i, 16)][...] = one16

    @pl.loop(0, RPT, step=16)
    def _(i):
      buf_v.at[pl.ds(i, 16)][...] = zero16

    pltpu.sync_copy(buf_v, acc.at[pl.ds(s * RPT, RPT)])
    pltpu.make_async_copy(e_hbm.at[wid], idx_v, isem).wait()
    plsc.subcore_barrier()

    ones = ones_v.at[pl.ds(0, CK)]

    @pl.loop(0, CH, step=10)
    def _(j):
      for t in range(10):
        @pl.when(j > 0)
        def _():
          pltpu.make_async_copy(ones, acc.at[pl.ds(0, CK)], ssem[t]).wait()

        pltpu.async_copy(ones, acc.at[idx_v.at[j + t, 1]], ssem[t], add=True)

    for t in range(10):
      pltpu.make_async_copy(ones, acc.at[pl.ds(0, CK)], ssem[t]).wait()

    plsc.subcore_barrier()
    pltpu.sync_copy(acc.at[pl.ds(s * RPT, RPT)], buf_v)
    pltpu.sync_copy(buf_v, out_hbm.at[c, pl.ds(s * RPT, RPT)])

  return _sc_degree


@functools.cache
def _sc_aggregate_kernel():
  mesh = plsc.VectorSubcoreMesh(core_axis_name="c", subcore_axis_name="s")

  @functools.partial(
      pl.kernel,
      out_type=jax.ShapeDtypeStruct((NC, NP, D), jnp.float32),
      mesh=mesh,
      scratch_types=[pltpu.VMEM_SHARED((NP, D), jnp.float32)]
      + [pltpu.VMEM((CK, D), jnp.float32) for _ in range(NBUF)]
      + [pltpu.VMEM((2, CK), jnp.int32) for _ in range(NIDX)]
      + [pltpu.SemaphoreType.DMA for _ in range(2 * NBUF + NIDX + 1)],
  )
  def _sc_aggregate(y_hbm, e_hbm, out_hbm, acc, *scr):
    rows = scr[:NBUF]
    slots = scr[NBUF:NBUF + NIDX]
    gsem = scr[NBUF + NIDX:2 * NBUF + NIDX]
    ssem = scr[2 * NBUF + NIDX:3 * NBUF + NIDX]
    isem = scr[3 * NBUF + NIDX:3 * NBUF + 2 * NIDX]
    zsem = scr[-1]
    c = lax.axis_index("c")
    s = lax.axis_index("s")
    wid = s * NC + c

    zero16 = jnp.zeros((16,), jnp.float32)

    @pl.loop(0, CK)
    def _(r):
      @pl.loop(0, D, step=16)
      def _(q):
        rows[0].at[r, pl.ds(q, 16)][...] = zero16

    for off, ln in _PIECES:
      pltpu.async_copy(rows[0].at[pl.ds(0, ln)],
                       acc.at[pl.ds(s * RPT + off, ln)], zsem)

    for k in range(NIDX):
      pltpu.async_copy(e_hbm.at[wid, k], slots[k], isem[k])
    for off, ln in _PIECES:
      pltpu.make_async_copy(rows[0].at[pl.ds(0, ln)],
                            acc.at[pl.ds(s * RPT, ln)], zsem).wait()
    for k in range(NBUF):
      pltpu.make_async_copy(e_hbm.at[wid, k], slots[k], isem[k]).wait()

    for b in range(NBUF):
      pltpu.async_copy(y_hbm.at[slots[b].at[0]], rows[b], gsem[b])

    plsc.subcore_barrier()

    @pl.loop(0, CH, step=2 * NBUF)
    def _(j):
      for t in range(NBUF):
        b = t
        pltpu.make_async_copy(y_hbm.at[slots[t].at[0]], rows[b],
                              gsem[b]).wait()
        pltpu.async_copy(rows[b], acc.at[slots[t].at[1]], ssem[b], add=True)
      for t in range(NBUF):
        b = t
        pltpu.make_async_copy(rows[b], acc.at[slots[t].at[1]], ssem[b]).wait()

        @pl.when(j + NIDX + t < CH)
        def _():
          pltpu.async_copy(e_hbm.at[wid, j + NIDX + t], slots[t], isem[t])

        pltpu.make_async_copy(e_hbm.at[wid, 0], slots[t + NBUF],
                              isem[t + NBUF]).wait()
        pltpu.async_copy(y_hbm.at[slots[t + NBUF].at[0]], rows[b], gsem[b])
      for t in range(NBUF, 2 * NBUF):
        b = t - NBUF
        pltpu.make_async_copy(y_hbm.at[slots[t].at[0]], rows[b],
                              gsem[b]).wait()
        pltpu.async_copy(rows[b], acc.at[slots[t].at[1]], ssem[b], add=True)
      for t in range(NBUF, 2 * NBUF):
        b = t - NBUF
        pltpu.make_async_copy(rows[b], acc.at[slots[t].at[1]], ssem[b]).wait()

        @pl.when(j + NIDX + t < CH)
        def _():
          pltpu.async_copy(e_hbm.at[wid, j + NIDX + t], slots[t], isem[t])

        @pl.when(j + NIDX + b < CH)
        def _():
          pltpu.make_async_copy(e_hbm.at[wid, 0], slots[b], isem[b]).wait()
          pltpu.async_copy(y_hbm.at[slots[b].at[0]], rows[b], gsem[b])

    plsc.subcore_barrier()

    for i, (off, ln) in enumerate(_PIECES):
      b = i % 2
      if i >= 2:
        po, pl_ = _PIECES[i - 2]
        pltpu.make_async_copy(rows[b].at[pl.ds(0, pl_)],
                              out_hbm.at[c, pl.ds(s * RPT, pl_)],
                              gsem[b]).wait()
      pltpu.sync_copy(acc.at[pl.ds(s * RPT + off, ln)],
                      rows[b].at[pl.ds(0, ln)])
      pltpu.async_copy(rows[b].at[pl.ds(0, ln)],
                       out_hbm.at[c, pl.ds(s * RPT + off, ln)], gsem[b])
    for i in range(max(0, len(_PIECES) - 2), len(_PIECES)):
      b = i % 2
      off, ln = _PIECES[i]
      pltpu.make_async_copy(rows[b].at[pl.ds(0, ln)],
                            out_hbm.at[c, pl.ds(s * RPT, ln)],
                            gsem[b]).wait()

  return _sc_aggregate


def _tc_first_body(x_ref, w_ref, degp_ref, y_ref, dinv_ref):
    deg = degp_ref[0, 0:N, :] + degp_ref[1, 0:N, :] + 1.0
    dinv = lax.rsqrt(deg)
    xw = jnp.dot(x_ref[...], w_ref[...], preferred_element_type=jnp.float32)
    y_ref[0:N, :] = dinv * xw
    y_ref[N:NP, :] = jnp.zeros((NP - N, D), jnp.float32)
    dinv_ref[...] = dinv


def _tc_first(x, w1, degp):
    return pl.pallas_call(
        _tc_first_body,
        out_shape=(
            jax.ShapeDtypeStruct((NP, D), jnp.float32),
            jax.ShapeDtypeStruct((N, 1), jnp.float32),
        ),
    )(x, w1, degp)


def _bn_core(sp_ref, y_ref, dinv_ref, b_ref, g_ref, be_ref):
    s = sp_ref[0, 0:N, :] + sp_ref[1, 0:N, :]
    dinv = dinv_ref[...]
    t = dinv * (s + y_ref[0:N, :]) + b_ref[...]
    m = jnp.mean(t, axis=0, keepdims=True)
    v = jnp.mean(t * t, axis=0, keepdims=True) - m * m
    return g_ref[...] * (t - m) * lax.rsqrt(v + EPS) + be_ref[...]


def _tc_mid_body(sp_ref, y_ref, dinv_ref, b_ref, g_ref, be_ref, w_ref, yn_ref):
    h = jnp.maximum(_bn_core(sp_ref, y_ref, dinv_ref, b_ref, g_ref, be_ref), 0.0)
    xw = jnp.dot(h, w_ref[...], preferred_element_type=jnp.float32)
    yn_ref[0:N, :] = dinv_ref[...] * xw
    yn_ref[N:NP, :] = jnp.zeros((NP - N, D), jnp.float32)


def _tc_mid(sp, y, dinv, b, g, be, w_next):
    return pl.pallas_call(
        _tc_mid_body,
        out_shape=jax.ShapeDtypeStruct((NP, D), jnp.float32),
    )(sp, y, dinv, b.reshape(1, D), g.reshape(1, D), be.reshape(1, D), w_next)


def _tc_last_body(sp_ref, y_ref, dinv_ref, b_ref, g_ref, be_ref, o_ref):
    o_ref[...] = _bn_core(sp_ref, y_ref, dinv_ref, b_ref, g_ref, be_ref)


def _tc_last(sp, y, dinv, b, g, be):
    return pl.pallas_call(
        _tc_last_body,
        out_shape=jax.ShapeDtypeStruct((N, D), jnp.float32),
    )(sp, y, dinv, b.reshape(1, D), g.reshape(1, D), be.reshape(1, D))


def kernel(x, edge_index, W1, b1, g1, be1, W2, b2, g2, be2,
           W3, b3, g3, be3, W4, b4, g4, be4):
    ei = edge_index.astype(jnp.int32)
    npad = EPAD - E
    pad_idx = N + (jnp.arange(npad, dtype=jnp.int32) % (NP - N))
    src = jnp.concatenate([ei[0], pad_idx])
    dst = jnp.concatenate([ei[1], pad_idx])
    edges = jnp.stack(
        [src.reshape(NW, CH, CK), dst.reshape(NW, CH, CK)], axis=2)

    _sc_degree = _sc_degree_kernel()
    _sc_aggregate = _sc_aggregate_kernel()
    degp = _sc_degree(edges).reshape(NC, NP, 1)
    y, dinv = _tc_first(x, W1, degp)
    sp = _sc_aggregate(y, edges)
    y = _tc_mid(sp, y, dinv, b1, g1, be1, W2)
    sp = _sc_aggregate(y, edges)
    y = _tc_mid(sp, y, dinv, b2, g2, be2, W3)
    sp = _sc_aggregate(y, edges)
    y = _tc_mid(sp, y, dinv, b3, g3, be3, W4)
    sp = _sc_aggregate(y, edges)
    return _tc_last(sp, y, dinv, b4, g4, be4)
